# baseline (speedup 1.0000x reference)
import functools

import jax
import jax.numpy as jnp
from jax import lax
from jax.experimental import pallas as pl
from jax.experimental.pallas import tpu as pltpu
from jax.experimental.pallas import tpu_sc as plsc

N_NODES = 10000
N_EDGES = 160000
NODE_DIM = 32
EDGE_DIM = 16
H1, H2 = 64, 128
BN_EPS = 1e-5

NC, NS = 2, 16
NW = NC * NS
GROUPS = 8
GROUP = N_EDGES // (NW * GROUPS)
N_PAD = 10240
ROWS_PER_TILE = N_PAD // NS
CNT_W = 16


@functools.lru_cache(maxsize=None)
def _sc_mesh():
    return plsc.VectorSubcoreMesh(
        core_axis_name="c", subcore_axis_name="s",
        num_cores=NC, num_subcores=NS)


def _gather_body(ns_hbm, src_hbm, out_hbm, idx_v, rows0, rows1,
                 gsem0, gsem1, wsem0, wsem1):
    c = lax.axis_index("c")
    s = lax.axis_index("s")
    wid = c * NS + s
    ebase = wid * (GROUPS * GROUP)
    pltpu.sync_copy(src_hbm.at[wid], idx_v)
    rows = (rows0, rows1)
    gsem = (gsem0, gsem1)
    wsem = (wsem0, wsem1)
    gcp = [None] * GROUPS
    wcp = [None] * GROUPS
    gcp[0] = pltpu.async_copy(ns_hbm.at[idx_v.at[0]], rows0, gsem0)
    for g in range(GROUPS):
        b = g % 2
        if g + 1 < GROUPS:
            b2 = (g + 1) % 2
            if g >= 1:
                wcp[g - 1].wait()
            gcp[g + 1] = pltpu.async_copy(ns_hbm.at[idx_v.at[g + 1]],
                                          rows[b2], gsem[b2])
        gcp[g].wait()
        wcp[g] = pltpu.async_copy(
            rows[b],
            out_hbm.at[pl.ds(ebase + g * GROUP, GROUP), pl.ds(0, NODE_DIM)],
            wsem[b])
    wcp[GROUPS - 2].wait()
    wcp[GROUPS - 1].wait()


@functools.lru_cache(maxsize=None)
def _gather_kernel():
    return pl.kernel(
        _gather_body,
        out_type=jax.ShapeDtypeStruct((N_EDGES, 128), jnp.bfloat16),
        mesh=_sc_mesh(),
        compiler_params=pltpu.CompilerParams(use_tc_tiling_on_sc=False),
        scratch_types=[
            pltpu.VMEM((GROUPS, GROUP), jnp.int32),
            pltpu.VMEM((GROUP, NODE_DIM), jnp.bfloat16),
            pltpu.VMEM((GROUP, NODE_DIM), jnp.bfloat16),
            pltpu.SemaphoreType.DMA,
            pltpu.SemaphoreType.DMA,
            pltpu.SemaphoreType.DMA,
            pltpu.SemaphoreType.DMA,
        ],
    )


BLK = 3200


def _mlp_body(e_ref, x_ref, w1_ref, b1_ref, w2_ref, b2_ref, w3_ref, b3_ref,
              r_ref, s_ref, o_ref):
    h = jnp.dot(e_ref[...], w1_ref[...], preferred_element_type=jnp.float32)
    h = jnp.maximum(h + b1_ref[...], 0.0)
    h = jnp.dot(h, w2_ref[...], preferred_element_type=jnp.float32)
    h = jnp.maximum(h + b2_ref[...], 0.0)
    z = jnp.dot(h.astype(jnp.bfloat16), w3_ref[...],
                preferred_element_type=jnp.float32).astype(jnp.bfloat16)
    a = jnp.maximum(z + b3_ref[...], jnp.bfloat16(0))
    xrep = jnp.dot(x_ref[:, 0:NODE_DIM], r_ref[...],
                   preferred_element_type=jnp.float32).astype(jnp.bfloat16)
    o_ref[:, 0:NODE_DIM] = jnp.dot(xrep * a, s_ref[...],
                                   preferred_element_type=jnp.float32)


def _run_mlp(edge, x_i, w1, b1, w2, b2, w3, b3):
    nd2 = NODE_DIM * NODE_DIM
    rk = lax.broadcasted_iota(jnp.int32, (NODE_DIM, nd2), 1)
    rd = lax.broadcasted_iota(jnp.int32, (NODE_DIM, nd2), 0)
    rm = (rk // NODE_DIM == rd).astype(jnp.bfloat16)
    sk = lax.broadcasted_iota(jnp.int32, (nd2, NODE_DIM), 0)
    sf = lax.broadcasted_iota(jnp.int32, (nd2, NODE_DIM), 1)
    sm = (sk % NODE_DIM == sf).astype(jnp.bfloat16)
    grid = (N_EDGES // BLK,)
    full = lambda shape: pl.BlockSpec(shape, lambda i: (0, 0))
    return pl.pallas_call(
        _mlp_body,
        grid=grid,
        in_specs=[
            pl.BlockSpec((BLK, EDGE_DIM), lambda i: (i, 0)),
            pl.BlockSpec((BLK, 128), lambda i: (i, 0)),
            full((EDGE_DIM, H1)),
            full((1, H1)),
            full((H1, H2)),
            full((1, H2)),
            full((H2, NODE_DIM * NODE_DIM)),
            full((1, NODE_DIM * NODE_DIM)),
            full((NODE_DIM, NODE_DIM * NODE_DIM)),
            full((NODE_DIM * NODE_DIM, NODE_DIM)),
        ],
        out_specs=pl.BlockSpec((BLK, 128), lambda i: (i, 0)),
        out_shape=jax.ShapeDtypeStruct((N_EDGES, 128), jnp.float32),
    )(edge, x_i, w1, b1, w2, b2, w3.astype(jnp.bfloat16),
      b3.astype(jnp.bfloat16), rm, sm)


def _scatter_body(msg_hbm, dst_hbm, z32_hbm, z16_hbm, ones_hbm,
                  sum_hbm, cnt_hbm,
                  idx_v, msg0, msg1, ones_v, acc_s, cnt_s, rsem0, rsem1):
    c = lax.axis_index("c")
    s = lax.axis_index("s")
    wid = c * NS + s
    ebase = wid * (GROUPS * GROUP)
    row0 = s * ROWS_PER_TILE
    pltpu.sync_copy(z32_hbm, acc_s.at[pl.ds(row0, ROWS_PER_TILE)])
    pltpu.sync_copy(z16_hbm, cnt_s.at[pl.ds(row0, ROWS_PER_TILE)])
    pltpu.sync_copy(ones_hbm, ones_v)
    pltpu.sync_copy(dst_hbm.at[wid], idx_v)
    plsc.subcore_barrier()
    msg = (msg0, msg1)
    rsem = (rsem0, rsem1)
    rcp = [None] * GROUPS
    rcp[0] = pltpu.async_copy(
        msg_hbm.at[pl.ds(ebase, GROUP), pl.ds(0, NODE_DIM)], msg0, rsem0)
    for g in range(GROUPS):
        b = g % 2
        if g + 1 < GROUPS:
            b2 = (g + 1) % 2
            rcp[g + 1] = pltpu.async_copy(
                msg_hbm.at[pl.ds(ebase + (g + 1) * GROUP, GROUP),
                           pl.ds(0, NODE_DIM)],
                msg[b2], rsem[b2])
        rcp[g].wait()
        pltpu.sync_copy(msg[b], acc_s.at[idx_v.at[g]], add=True)
        pltpu.sync_copy(ones_v, cnt_s.at[idx_v.at[g]], add=True)
    plsc.subcore_barrier()
    out0 = c * N_PAD + row0
    pltpu.sync_copy(acc_s.at[pl.ds(row0, ROWS_PER_TILE)],
                    sum_hbm.at[pl.ds(out0, ROWS_PER_TILE)])
    pltpu.sync_copy(cnt_s.at[pl.ds(row0, ROWS_PER_TILE)],
                    cnt_hbm.at[pl.ds(out0, ROWS_PER_TILE)])


@functools.lru_cache(maxsize=None)
def _scatter_kernel():
    return pl.kernel(
        _scatter_body,
        out_type=(
            jax.ShapeDtypeStruct((NC * N_PAD, NODE_DIM), jnp.float32),
            jax.ShapeDtypeStruct((NC * N_PAD, CNT_W), jnp.float32),
        ),
        mesh=_sc_mesh(),
        compiler_params=pltpu.CompilerParams(use_tc_tiling_on_sc=False),
        scratch_types=[
            pltpu.VMEM((GROUPS, GROUP), jnp.int32),
            pltpu.VMEM((GROUP, NODE_DIM), jnp.float32),
            pltpu.VMEM((GROUP, NODE_DIM), jnp.float32),
            pltpu.VMEM((GROUP, CNT_W), jnp.float32),
            pltpu.VMEM_SHARED((N_PAD, NODE_DIM), jnp.float32),
            pltpu.VMEM_SHARED((N_PAD, CNT_W), jnp.float32),
            pltpu.SemaphoreType.DMA,
            pltpu.SemaphoreType.DMA,
        ],
    )


def _fin_body(sum_ref, cnt_ref, bias_ref, o_ref):
    total = sum_ref[0, :N_NODES, :] + sum_ref[1, :N_NODES, :]
    cnt = cnt_ref[0, :N_NODES, 0:1] + cnt_ref[1, :N_NODES, 0:1]
    o_ref[...] = total / jnp.maximum(cnt, 1.0) + bias_ref[...]


def _finalize(sums, cnts, bias2d):
    return pl.pallas_call(
        _fin_body,
        in_specs=[
            pl.BlockSpec((NC, N_PAD, NODE_DIM), lambda: (0, 0, 0)),
            pl.BlockSpec((NC, N_PAD, CNT_W), lambda: (0, 0, 0)),
            pl.BlockSpec((1, NODE_DIM), lambda: (0, 0)),
        ],
        out_specs=pl.BlockSpec((N_NODES, NODE_DIM), lambda: (0, 0)),
        out_shape=jax.ShapeDtypeStruct((N_NODES, NODE_DIM), jnp.float32),
    )(sums, cnts, bias2d)


def kernel(node_states, edge_index, edge, W1, g1, b1, W2, g2, b2, W3, g3, b3,
           bias):
    inv = 1.0 / jnp.sqrt(1.0 + BN_EPS)
    w1 = (W1 * (g1 * inv)[:, None]).T
    w2 = (W2 * (g2 * inv)[:, None]).T
    w3 = (W3 * (g3 * inv)[:, None]).T
    src = edge_index[:, 0].reshape(NW, GROUPS, GROUP)
    dst = edge_index[:, 1].reshape(NW, GROUPS, GROUP)

    x_i = _gather_kernel()(node_states.astype(jnp.bfloat16), src)
    msgs = _run_mlp(edge, x_i, w1, b1.reshape(1, H1), w2, b2.reshape(1, H2),
                    w3, b3.reshape(1, NODE_DIM * NODE_DIM))
    z32 = jnp.zeros((ROWS_PER_TILE, NODE_DIM), jnp.float32)
    z16 = jnp.zeros((ROWS_PER_TILE, CNT_W), jnp.float32)
    ones = jnp.ones((GROUP, CNT_W), jnp.float32)
    sums, cnts = _scatter_kernel()(msgs, dst, z32, z16, ones)
    return _finalize(sums.reshape(NC, N_PAD, NODE_DIM),
                     cnts.reshape(NC, N_PAD, CNT_W), bias.reshape(1, NODE_DIM))


# speedup vs baseline: 1.3463x; 1.3463x over previous
import functools

import jax
import jax.numpy as jnp
from jax import lax
from jax.experimental import pallas as pl
from jax.experimental.pallas import tpu as pltpu
from jax.experimental.pallas import tpu_sc as plsc

N_NODES = 10000
N_EDGES = 160000
NODE_DIM = 32
EDGE_DIM = 16
H1, H2 = 64, 128
BN_EPS = 1e-5

NC, NS = 2, 16
NW = NC * NS
GROUPS = 8
GROUP = N_EDGES // (NW * GROUPS)
N_PAD = 10240
ROWS_PER_TILE = N_PAD // NS
CNT_W = 16


@functools.lru_cache(maxsize=None)
def _sc_mesh():
    return plsc.VectorSubcoreMesh(
        core_axis_name="c", subcore_axis_name="s",
        num_cores=NC, num_subcores=NS)


def _gather_body(ns_hbm, src_hbm, out_hbm, idx_v, rows0, rows1,
                 gsem0, gsem1, wsem0, wsem1):
    c = lax.axis_index("c")
    s = lax.axis_index("s")
    wid = c * NS + s
    ebase = wid * (GROUPS * GROUP)
    pltpu.sync_copy(src_hbm.at[wid], idx_v)
    rows = (rows0, rows1)
    gsem = (gsem0, gsem1)
    wsem = (wsem0, wsem1)
    gcp = [None] * GROUPS
    wcp = [None] * GROUPS
    gcp[0] = pltpu.async_copy(ns_hbm.at[idx_v.at[0]], rows0, gsem0)
    for g in range(GROUPS):
        b = g % 2
        if g + 1 < GROUPS:
            b2 = (g + 1) % 2
            if g >= 1:
                wcp[g - 1].wait()
            gcp[g + 1] = pltpu.async_copy(ns_hbm.at[idx_v.at[g + 1]],
                                          rows[b2], gsem[b2])
        gcp[g].wait()
        wcp[g] = pltpu.async_copy(
            rows[b],
            out_hbm.at[pl.ds(ebase + g * GROUP, GROUP), pl.ds(0, NODE_DIM)],
            wsem[b])
    wcp[GROUPS - 2].wait()
    wcp[GROUPS - 1].wait()


@functools.lru_cache(maxsize=None)
def _gather_kernel():
    return pl.kernel(
        _gather_body,
        out_type=jax.ShapeDtypeStruct((N_EDGES, 128), jnp.float32),
        mesh=_sc_mesh(),
        compiler_params=pltpu.CompilerParams(use_tc_tiling_on_sc=False),
        scratch_types=[
            pltpu.VMEM((GROUPS, GROUP), jnp.int32),
            pltpu.VMEM((GROUP, NODE_DIM), jnp.float32),
            pltpu.VMEM((GROUP, NODE_DIM), jnp.float32),
            pltpu.SemaphoreType.DMA,
            pltpu.SemaphoreType.DMA,
            pltpu.SemaphoreType.DMA,
            pltpu.SemaphoreType.DMA,
        ],
    )


BLK = 3200


def _mlp_body(e_ref, x_ref, w1_ref, b1_ref, w2_ref, b2_ref, w3_ref, b3_ref,
              r_ref, s_ref, o_ref):
    h = jnp.dot(e_ref[...], w1_ref[...], preferred_element_type=jnp.float32)
    h = jnp.maximum(h + b1_ref[...], 0.0)
    h = jnp.dot(h, w2_ref[...], preferred_element_type=jnp.float32)
    h = jnp.maximum(h + b2_ref[...], 0.0)
    z = jnp.dot(h.astype(jnp.bfloat16), w3_ref[...],
                preferred_element_type=jnp.float32).astype(jnp.bfloat16)
    a = jnp.maximum(z + b3_ref[...], jnp.bfloat16(0))
    xrep = jnp.dot(x_ref[:, 0:NODE_DIM].astype(jnp.bfloat16), r_ref[...],
                   preferred_element_type=jnp.float32).astype(jnp.bfloat16)
    o_ref[:, 0:NODE_DIM] = jnp.dot(xrep * a, s_ref[...],
                                   preferred_element_type=jnp.float32)


def _run_mlp(edge, x_i, w1, b1, w2, b2, w3, b3):
    nd2 = NODE_DIM * NODE_DIM
    rk = lax.broadcasted_iota(jnp.int32, (NODE_DIM, nd2), 1)
    rd = lax.broadcasted_iota(jnp.int32, (NODE_DIM, nd2), 0)
    rm = (rk // NODE_DIM == rd).astype(jnp.bfloat16)
    sk = lax.broadcasted_iota(jnp.int32, (nd2, NODE_DIM), 0)
    sf = lax.broadcasted_iota(jnp.int32, (nd2, NODE_DIM), 1)
    sm = (sk % NODE_DIM == sf).astype(jnp.bfloat16)
    grid = (N_EDGES // BLK,)
    full = lambda shape: pl.BlockSpec(shape, lambda i: (0, 0))
    return pl.pallas_call(
        _mlp_body,
        grid=grid,
        in_specs=[
            pl.BlockSpec((BLK, EDGE_DIM), lambda i: (i, 0)),
            pl.BlockSpec((BLK, 128), lambda i: (i, 0)),
            full((EDGE_DIM, H1)),
            full((1, H1)),
            full((H1, H2)),
            full((1, H2)),
            full((H2, NODE_DIM * NODE_DIM)),
            full((1, NODE_DIM * NODE_DIM)),
            full((NODE_DIM, NODE_DIM * NODE_DIM)),
            full((NODE_DIM * NODE_DIM, NODE_DIM)),
        ],
        out_specs=pl.BlockSpec((BLK, 128), lambda i: (i, 0)),
        out_shape=jax.ShapeDtypeStruct((N_EDGES, 128), jnp.float32),
    )(edge, x_i, w1, b1, w2, b2, w3.astype(jnp.bfloat16),
      b3.astype(jnp.bfloat16), rm, sm)


def _scatter_body(msg_hbm, dst_hbm, z32_hbm, z16_hbm, ones_hbm,
                  sum_hbm, cnt_hbm,
                  idx_v, msg0, msg1, ones_v, acc_s, cnt_s, rsem0, rsem1):
    c = lax.axis_index("c")
    s = lax.axis_index("s")
    wid = c * NS + s
    ebase = wid * (GROUPS * GROUP)
    row0 = s * ROWS_PER_TILE
    pltpu.sync_copy(z32_hbm, acc_s.at[pl.ds(row0, ROWS_PER_TILE)])
    pltpu.sync_copy(z16_hbm, cnt_s.at[pl.ds(row0, ROWS_PER_TILE)])
    pltpu.sync_copy(ones_hbm, ones_v)
    pltpu.sync_copy(dst_hbm.at[wid], idx_v)
    plsc.subcore_barrier()
    msg = (msg0, msg1)
    rsem = (rsem0, rsem1)
    rcp = [None] * GROUPS
    rcp[0] = pltpu.async_copy(
        msg_hbm.at[pl.ds(ebase, GROUP), pl.ds(0, NODE_DIM)], msg0, rsem0)
    for g in range(GROUPS):
        b = g % 2
        if g + 1 < GROUPS:
            b2 = (g + 1) % 2
            rcp[g + 1] = pltpu.async_copy(
                msg_hbm.at[pl.ds(ebase + (g + 1) * GROUP, GROUP),
                           pl.ds(0, NODE_DIM)],
                msg[b2], rsem[b2])
        rcp[g].wait()
        pltpu.sync_copy(msg[b], acc_s.at[idx_v.at[g]], add=True)
        pltpu.sync_copy(ones_v, cnt_s.at[idx_v.at[g]], add=True)
    plsc.subcore_barrier()
    out0 = c * N_PAD + row0
    pltpu.sync_copy(acc_s.at[pl.ds(row0, ROWS_PER_TILE)],
                    sum_hbm.at[pl.ds(out0, ROWS_PER_TILE)])
    pltpu.sync_copy(cnt_s.at[pl.ds(row0, ROWS_PER_TILE)],
                    cnt_hbm.at[pl.ds(out0, ROWS_PER_TILE)])


@functools.lru_cache(maxsize=None)
def _scatter_kernel():
    return pl.kernel(
        _scatter_body,
        out_type=(
            jax.ShapeDtypeStruct((NC * N_PAD, NODE_DIM), jnp.float32),
            jax.ShapeDtypeStruct((NC * N_PAD, CNT_W), jnp.float32),
        ),
        mesh=_sc_mesh(),
        compiler_params=pltpu.CompilerParams(use_tc_tiling_on_sc=False),
        scratch_types=[
            pltpu.VMEM((GROUPS, GROUP), jnp.int32),
            pltpu.VMEM((GROUP, NODE_DIM), jnp.float32),
            pltpu.VMEM((GROUP, NODE_DIM), jnp.float32),
            pltpu.VMEM((GROUP, CNT_W), jnp.float32),
            pltpu.VMEM_SHARED((N_PAD, NODE_DIM), jnp.float32),
            pltpu.VMEM_SHARED((N_PAD, CNT_W), jnp.float32),
            pltpu.SemaphoreType.DMA,
            pltpu.SemaphoreType.DMA,
        ],
    )


def _fin_body(sum_ref, cnt_ref, bias_ref, o_ref):
    total = sum_ref[0, :N_NODES, :] + sum_ref[1, :N_NODES, :]
    cnt = cnt_ref[0, :N_NODES, 0:1] + cnt_ref[1, :N_NODES, 0:1]
    o_ref[...] = total / jnp.maximum(cnt, 1.0) + bias_ref[...]


def _finalize(sums, cnts, bias2d):
    return pl.pallas_call(
        _fin_body,
        in_specs=[
            pl.BlockSpec((NC, N_PAD, NODE_DIM), lambda: (0, 0, 0)),
            pl.BlockSpec((NC, N_PAD, CNT_W), lambda: (0, 0, 0)),
            pl.BlockSpec((1, NODE_DIM), lambda: (0, 0)),
        ],
        out_specs=pl.BlockSpec((N_NODES, NODE_DIM), lambda: (0, 0)),
        out_shape=jax.ShapeDtypeStruct((N_NODES, NODE_DIM), jnp.float32),
    )(sums, cnts, bias2d)


def kernel(node_states, edge_index, edge, W1, g1, b1, W2, g2, b2, W3, g3, b3,
           bias):
    inv = 1.0 / jnp.sqrt(1.0 + BN_EPS)
    w1 = (W1 * (g1 * inv)[:, None]).T
    w2 = (W2 * (g2 * inv)[:, None]).T
    w3 = (W3 * (g3 * inv)[:, None]).T
    src = edge_index[:, 0].reshape(NW, GROUPS, GROUP)
    dst = edge_index[:, 1].reshape(NW, GROUPS, GROUP)

    x_i = _gather_kernel()(node_states, src)
    msgs = _run_mlp(edge, x_i, w1, b1.reshape(1, H1), w2, b2.reshape(1, H2),
                    w3, b3.reshape(1, NODE_DIM * NODE_DIM))
    z32 = jnp.zeros((ROWS_PER_TILE, NODE_DIM), jnp.float32)
    z16 = jnp.zeros((ROWS_PER_TILE, CNT_W), jnp.float32)
    ones = jnp.ones((GROUP, CNT_W), jnp.float32)
    sums, cnts = _scatter_kernel()(msgs, dst, z32, z16, ones)
    return _finalize(sums.reshape(NC, N_PAD, NODE_DIM),
                     cnts.reshape(NC, N_PAD, CNT_W), bias.reshape(1, NODE_DIM))
